# baseline (speedup 1.0000x reference)
import functools

import jax
import jax.numpy as jnp
from jax import lax
from jax.experimental import pallas as pl
from jax.experimental.pallas import tpu as pltpu
from jax.experimental.pallas import tpu_sc as plsc

E = 8
TOPK = 2
D = 1024
DFF = 1024
SDFF = 2048
T = 2048
NPAIR = T * TOPK
BLK = 256
PMAX = NPAIR + E * BLK
NB = PMAX // BLK
NW = 32
BEXP_COLS = 64

_f32 = jnp.float32
_i32 = jnp.int32


def _plan_body(logits_ref, res_ref, bexp_ref):
    logits = logits_ref[...]
    m = jnp.max(logits, axis=1, keepdims=True)
    p = jnp.exp(logits - m)
    scores = p / jnp.sum(p, axis=1, keepdims=True)

    eidx = lax.broadcasted_iota(_i32, (T, E), 1).astype(_f32)
    m1 = jnp.max(scores, axis=1, keepdims=True)
    i1 = jnp.min(jnp.where(scores >= m1, eidx, float(E)), axis=1,
                 keepdims=True)
    c1 = (eidx == i1).astype(_f32)
    rest = jnp.where(c1 > 0, -jnp.inf, scores)
    m2 = jnp.max(rest, axis=1, keepdims=True)
    i2 = jnp.min(jnp.where(rest >= m2, eidx, float(E)), axis=1,
                 keepdims=True)
    c2 = (eidx == i2).astype(_f32)

    denom = m1 + m2 + 1e-20
    w1 = m1 / denom
    w2 = m2 / denom

    ones_t = jnp.ones((T, 1), _f32)
    ct = c1 + c2
    counts_col = lax.dot_general(ct, ones_t, (((0,), (0,)), ((), ())),
                                 precision=lax.Precision.HIGHEST)
    rup_col = jnp.ceil(counts_col / BLK) * BLK
    l8 = (lax.broadcasted_iota(_i32, (E, E), 1)
          < lax.broadcasted_iota(_i32, (E, E), 0)).astype(_f32)
    astart_col = lax.dot_general(l8, rup_col, (((1,), (0,)), ((), ())),
                                 precision=lax.Precision.HIGHEST)
    aend_col = astart_col + rup_col

    def rowb(v_col):
        return lax.dot_general(ones_t, v_col, (((1,), (1,)), ((), ())),
                               precision=lax.Precision.HIGHEST)

    ltri = (lax.broadcasted_iota(_i32, (T, T), 1)
            < lax.broadcasted_iota(_i32, (T, T), 0)).astype(_f32)
    cnt1 = lax.dot_general(ltri, c1, (((1,), (0,)), ((), ())))
    cnt2 = lax.dot_general(ltri, c2, (((1,), (0,)), ((), ())))
    c1tot_col = lax.dot_general(c1, ones_t, (((0,), (0,)), ((), ())),
                                precision=lax.Precision.HIGHEST)

    base = rowb(astart_col)
    pos1 = jnp.sum(c1 * (base + cnt1), axis=1, keepdims=True)
    pos2 = jnp.sum(c2 * (base + rowb(c1tot_col) + cnt2), axis=1,
                   keepdims=True)

    zero = jnp.zeros((T, E), _f32)
    res = (jnp.where(eidx == 0, w1, zero)
           + jnp.where(eidx == 1, w2, zero)
           + jnp.where(eidx == 2, pos1, zero)
           + jnp.where(eidx == 3, pos2, zero))
    res_ref[...] = res

    ivals = lax.broadcasted_iota(_i32, (E, BEXP_COLS), 1).astype(_f32) * BLK
    ind = (ivals >= aend_col).astype(_f32)
    be_row = jnp.sum(ind, axis=0, keepdims=True)
    last_e = jnp.max(
        jnp.where(counts_col > 0,
                  lax.broadcasted_iota(_i32, (E, 1), 0).astype(_f32), 0.0))
    used = jnp.sum(rup_col) / BLK
    be_clamped = jnp.minimum(be_row, last_e)
    out8 = jnp.broadcast_to(be_clamped, (E, BEXP_COLS))
    colid = lax.broadcasted_iota(_i32, (E, BEXP_COLS), 1).astype(_f32)
    bexp_ref[...] = jnp.where(colid == NB, used, out8)


def _plan_call(logits):
    return pl.pallas_call(
        _plan_body,
        out_shape=[
            jax.ShapeDtypeStruct((T, E), _f32),
            jax.ShapeDtypeStruct((E, BEXP_COLS), _f32),
        ],
    )(logits)


@functools.cache
def _build_dispatch():
    mesh = plsc.VectorSubcoreMesh(core_axis_name="c", subcore_axis_name="s",
                                  num_cores=2, num_subcores=16)

    @functools.partial(
        pl.kernel,
        out_type=jax.ShapeDtypeStruct((PMAX, D), _f32),
        mesh=mesh,
        scratch_types=[
            pltpu.VMEM((2, 64), _i32),
            pltpu.VMEM((2, 64), _i32),
            pltpu.VMEM((64, D), _f32),
            pltpu.SemaphoreType.DMA,
            pltpu.SemaphoreType.DMA,
        ],
    )
    def dispatch(x_hbm, tok_hbm, pos_hbm, xs_hbm, tokv, posv, rows,
                 sem_g, sem_s):
        wid = lax.axis_index("s") * 2 + lax.axis_index("c")
        pltpu.sync_copy(tok_hbm.at[wid], tokv)
        pltpu.sync_copy(pos_hbm.at[wid], posv)
        for j in range(2):
            pltpu.async_copy(x_hbm.at[tokv.at[j]], rows, sem_g).wait()
            pltpu.async_copy(rows, xs_hbm.at[posv.at[j]], sem_s).wait()

    return dispatch


def _dispatch_call(x, tok_arr, pos_arr):
    return _build_dispatch()(x, tok_arr, pos_arr)


def _gffn_body(be_ref, xs_ref, wg_ref, wu_ref, wd_ref, out_ref):
    i = pl.program_id(0)

    @pl.when(i < be_ref[NB])
    def _():
        xb = xs_ref[...]
        g = lax.dot_general(xb, wg_ref[0], (((1,), (1,)), ((), ())),
                            preferred_element_type=_f32)
        u = lax.dot_general(xb, wu_ref[0], (((1,), (1,)), ((), ())),
                            preferred_element_type=_f32)
        h = g * jax.nn.sigmoid(g) * u
        out_ref[...] = lax.dot_general(h, wd_ref[0], (((1,), (1,)), ((), ())),
                                       preferred_element_type=_f32)


def _gffn_call(be, xs, w_gate, w_up, w_down):
    grid_spec = pltpu.PrefetchScalarGridSpec(
        num_scalar_prefetch=1,
        grid=(NB,),
        in_specs=[
            pl.BlockSpec((BLK, D), lambda i, be: (i, 0)),
            pl.BlockSpec((1, DFF, D), lambda i, be: (be[i], 0, 0)),
            pl.BlockSpec((1, DFF, D), lambda i, be: (be[i], 0, 0)),
            pl.BlockSpec((1, D, DFF), lambda i, be: (be[i], 0, 0)),
        ],
        out_specs=pl.BlockSpec((BLK, D), lambda i, be: (i, 0)),
    )
    return pl.pallas_call(
        _gffn_body,
        grid_spec=grid_spec,
        out_shape=jax.ShapeDtypeStruct((PMAX, D), _f32),
        compiler_params=pltpu.CompilerParams(
            dimension_semantics=("arbitrary",)),
    )(be, xs, w_gate, w_up, w_down)


_SBT = 256
_SBF = 1024


def _sffn_body(x_ref, sg_ref, su_ref, sd_ref, out_ref):
    f = pl.program_id(1)
    xb = x_ref[...]
    g = lax.dot_general(xb, sg_ref[...], (((1,), (1,)), ((), ())),
                        preferred_element_type=_f32)
    u = lax.dot_general(xb, su_ref[...], (((1,), (1,)), ((), ())),
                        preferred_element_type=_f32)
    h = g * jax.nn.sigmoid(g) * u
    o = lax.dot_general(h, sd_ref[...], (((1,), (1,)), ((), ())),
                        preferred_element_type=_f32)

    @pl.when(f == 0)
    def _():
        out_ref[...] = o

    @pl.when(f != 0)
    def _():
        out_ref[...] += o


def _sffn_call(x, s_gate, s_up, s_down):
    return pl.pallas_call(
        _sffn_body,
        grid=(T // _SBT, SDFF // _SBF),
        in_specs=[
            pl.BlockSpec((_SBT, D), lambda t, f: (t, 0)),
            pl.BlockSpec((_SBF, D), lambda t, f: (f, 0)),
            pl.BlockSpec((_SBF, D), lambda t, f: (f, 0)),
            pl.BlockSpec((D, _SBF), lambda t, f: (0, f)),
        ],
        out_specs=pl.BlockSpec((_SBT, D), lambda t, f: (t, 0)),
        out_shape=jax.ShapeDtypeStruct((T, D), _f32),
        compiler_params=pltpu.CompilerParams(
            dimension_semantics=("parallel", "arbitrary")),
    )(x, s_gate, s_up, s_down)


@functools.cache
def _build_gather():
    mesh = plsc.VectorSubcoreMesh(core_axis_name="c", subcore_axis_name="s",
                                  num_cores=2, num_subcores=16)

    @functools.partial(
        pl.kernel,
        out_type=jax.ShapeDtypeStruct((NPAIR, D), _f32),
        mesh=mesh,
        scratch_types=[
            pltpu.VMEM((2, 64), _i32),
            pltpu.VMEM((64, D), _f32),
            pltpu.SemaphoreType.DMA,
        ],
    )
    def gather(ys_hbm, pos_hbm, out_hbm, posv, rows, sem):
        wid = lax.axis_index("s") * 2 + lax.axis_index("c")
        pltpu.sync_copy(pos_hbm.at[wid], posv)
        for j in range(2):
            pltpu.async_copy(ys_hbm.at[posv.at[j]], rows, sem).wait()
            pltpu.sync_copy(rows, out_hbm.at[pl.ds(wid * 128 + j * 64, 64)])

    return gather


def _gather_call(ys, pos_arr):
    return _build_gather()(ys, pos_arr)


_CBT = 512


def _cmb_body(wts_ref, sh_ref, y1_ref, y2_ref, out_ref):
    w1 = wts_ref[:, 0:1]
    w2 = wts_ref[:, 1:2]
    out_ref[...] = sh_ref[...] + w1 * y1_ref[...] + w2 * y2_ref[...]


def _cmb_call(wts, sh, y1, y2):
    return pl.pallas_call(
        _cmb_body,
        grid=(T // _CBT,),
        in_specs=[
            pl.BlockSpec((_CBT, E), lambda t: (t, 0)),
            pl.BlockSpec((_CBT, D), lambda t: (t, 0)),
            pl.BlockSpec((_CBT, D), lambda t: (t, 0)),
            pl.BlockSpec((_CBT, D), lambda t: (t, 0)),
        ],
        out_specs=pl.BlockSpec((_CBT, D), lambda t: (t, 0)),
        out_shape=jax.ShapeDtypeStruct((T, D), _f32),
        compiler_params=pltpu.CompilerParams(
            dimension_semantics=("parallel",)),
    )(wts, sh, y1, y2)


def kernel(hidden_states, router_w, w_gate, w_up, w_down, s_gate, s_up,
           s_down):
    bsz, seq_len, h = hidden_states.shape
    x = hidden_states.reshape(T, D)

    logits = x @ router_w.T
    res, bexp = _plan_call(logits)
    w1 = res[:, 0]
    w2 = res[:, 1]
    pos1 = res[:, 2].astype(_i32)
    pos2 = res[:, 3].astype(_i32)
    be = bexp[0, :NB + 1].astype(_i32)

    tok = jnp.concatenate([jnp.arange(T, dtype=_i32),
                           jnp.arange(T, dtype=_i32)])
    pos = jnp.concatenate([pos1, pos2])
    xs = _dispatch_call(x, tok.reshape(NW, 2, 64), pos.reshape(NW, 2, 64))

    ys = _gffn_call(be, xs, w_gate, w_up, w_down)
    sh = _sffn_call(x, s_gate, s_up, s_down)

    yp = _gather_call(ys, pos.reshape(NW, 2, 64))
    y = _cmb_call(res, sh, yp[:T], yp[T:])
    return y.reshape(bsz, seq_len, h)


# speedup vs baseline: 1.2094x; 1.2094x over previous
import functools

import jax
import jax.numpy as jnp
from jax import lax
from jax.experimental import pallas as pl
from jax.experimental.pallas import tpu as pltpu
from jax.experimental.pallas import tpu_sc as plsc

E = 8
TOPK = 2
D = 1024
DFF = 1024
SDFF = 2048
T = 2048
NPAIR = T * TOPK
BLK = 256
PMAX = NPAIR + E * BLK
NB = PMAX // BLK
NW = 32
BEXP_COLS = 64

_f32 = jnp.float32
_i32 = jnp.int32


def _plan_body(logits_ref, res_ref, bexp_ref):
    logits = logits_ref[...]
    m = jnp.max(logits, axis=1, keepdims=True)
    p = jnp.exp(logits - m)
    scores = p / jnp.sum(p, axis=1, keepdims=True)

    eidx = lax.broadcasted_iota(_i32, (T, E), 1).astype(_f32)
    m1 = jnp.max(scores, axis=1, keepdims=True)
    i1 = jnp.min(jnp.where(scores >= m1, eidx, float(E)), axis=1,
                 keepdims=True)
    c1 = (eidx == i1).astype(_f32)
    rest = jnp.where(c1 > 0, -jnp.inf, scores)
    m2 = jnp.max(rest, axis=1, keepdims=True)
    i2 = jnp.min(jnp.where(rest >= m2, eidx, float(E)), axis=1,
                 keepdims=True)
    c2 = (eidx == i2).astype(_f32)

    denom = m1 + m2 + 1e-20
    w1 = m1 / denom
    w2 = m2 / denom

    ones_t = jnp.ones((T, 1), _f32)
    ct = c1 + c2
    counts_col = lax.dot_general(ct, ones_t, (((0,), (0,)), ((), ())),
                                 precision=lax.Precision.HIGHEST)
    rup_col = jnp.ceil(counts_col / BLK) * BLK
    l8 = (lax.broadcasted_iota(_i32, (E, E), 1)
          < lax.broadcasted_iota(_i32, (E, E), 0)).astype(_f32)
    astart_col = lax.dot_general(l8, rup_col, (((1,), (0,)), ((), ())),
                                 precision=lax.Precision.HIGHEST)
    aend_col = astart_col + rup_col

    def rowb(v_col):
        return lax.dot_general(ones_t, v_col, (((1,), (1,)), ((), ())),
                               precision=lax.Precision.HIGHEST)

    ltri = (lax.broadcasted_iota(_i32, (T, T), 1)
            < lax.broadcasted_iota(_i32, (T, T), 0)).astype(_f32)
    cnt1 = lax.dot_general(ltri, c1, (((1,), (0,)), ((), ())))
    cnt2 = lax.dot_general(ltri, c2, (((1,), (0,)), ((), ())))
    c1tot_col = lax.dot_general(c1, ones_t, (((0,), (0,)), ((), ())),
                                precision=lax.Precision.HIGHEST)

    base = rowb(astart_col)
    pos1 = jnp.sum(c1 * (base + cnt1), axis=1, keepdims=True)
    pos2 = jnp.sum(c2 * (base + rowb(c1tot_col) + cnt2), axis=1,
                   keepdims=True)

    zero = jnp.zeros((T, E), _f32)
    res = (jnp.where(eidx == 0, w1, zero)
           + jnp.where(eidx == 1, w2, zero)
           + jnp.where(eidx == 2, pos1, zero)
           + jnp.where(eidx == 3, pos2, zero))
    res_ref[...] = res

    ivals = lax.broadcasted_iota(_i32, (E, BEXP_COLS), 1).astype(_f32) * BLK
    ind = (ivals >= aend_col).astype(_f32)
    be_row = jnp.sum(ind, axis=0, keepdims=True)
    last_e = jnp.max(
        jnp.where(counts_col > 0,
                  lax.broadcasted_iota(_i32, (E, 1), 0).astype(_f32), 0.0))
    used = jnp.sum(rup_col) / BLK
    be_clamped = jnp.minimum(be_row, last_e)
    out8 = jnp.broadcast_to(be_clamped, (E, BEXP_COLS))
    colid = lax.broadcasted_iota(_i32, (E, BEXP_COLS), 1).astype(_f32)
    bexp_ref[...] = jnp.where(colid == NB, used, out8)


def _plan_call(logits):
    return pl.pallas_call(
        _plan_body,
        out_shape=[
            jax.ShapeDtypeStruct((T, E), _f32),
            jax.ShapeDtypeStruct((E, BEXP_COLS), _f32),
        ],
    )(logits)


@functools.cache
def _build_dispatch():
    mesh = plsc.VectorSubcoreMesh(core_axis_name="c", subcore_axis_name="s",
                                  num_cores=2, num_subcores=16)

    @functools.partial(
        pl.kernel,
        out_type=jax.ShapeDtypeStruct((PMAX, D), _f32),
        mesh=mesh,
        scratch_types=[
            pltpu.VMEM((2, 64), _i32),
            pltpu.VMEM((2, 64), _i32),
            pltpu.VMEM((64, D), _f32),
            pltpu.SemaphoreType.DMA,
            pltpu.SemaphoreType.DMA,
        ],
    )
    def dispatch(x_hbm, tok_hbm, pos_hbm, xs_hbm, tokv, posv, rows,
                 sem_g, sem_s):
        wid = lax.axis_index("s") * 2 + lax.axis_index("c")
        pltpu.sync_copy(tok_hbm.at[wid], tokv)
        pltpu.sync_copy(pos_hbm.at[wid], posv)
        for j in range(2):
            pltpu.async_copy(x_hbm.at[tokv.at[j]], rows, sem_g).wait()
            pltpu.async_copy(rows, xs_hbm.at[posv.at[j]], sem_s).wait()

    return dispatch


def _dispatch_call(x, tok_arr, pos_arr):
    return _build_dispatch()(x, tok_arr, pos_arr)


def _gffn_body(be_ref, xs_ref, wg_ref, wu_ref, wd_ref, out_ref):
    i = pl.program_id(0)

    @pl.when(i < be_ref[NB])
    def _():
        xb = xs_ref[...]
        g = lax.dot_general(xb, wg_ref[0], (((1,), (1,)), ((), ())),
                            preferred_element_type=_f32)
        u = lax.dot_general(xb, wu_ref[0], (((1,), (1,)), ((), ())),
                            preferred_element_type=_f32)
        h = g * jax.nn.sigmoid(g) * u
        out_ref[...] = lax.dot_general(h, wd_ref[0], (((1,), (1,)), ((), ())),
                                       preferred_element_type=_f32)


def _gffn_call(be, xs, w_gate, w_up, w_down):
    grid_spec = pltpu.PrefetchScalarGridSpec(
        num_scalar_prefetch=1,
        grid=(NB,),
        in_specs=[
            pl.BlockSpec((BLK, D), lambda i, be: (i, 0)),
            pl.BlockSpec((1, DFF, D), lambda i, be: (be[i], 0, 0)),
            pl.BlockSpec((1, DFF, D), lambda i, be: (be[i], 0, 0)),
            pl.BlockSpec((1, D, DFF), lambda i, be: (be[i], 0, 0)),
        ],
        out_specs=pl.BlockSpec((BLK, D), lambda i, be: (i, 0)),
    )
    return pl.pallas_call(
        _gffn_body,
        grid_spec=grid_spec,
        out_shape=jax.ShapeDtypeStruct((PMAX, D), _f32),
        compiler_params=pltpu.CompilerParams(
            dimension_semantics=("arbitrary",)),
    )(be, xs, w_gate, w_up, w_down)


_SBT = 512


def _sffn_body(x_ref, sg_ref, su_ref, sd_ref, out_ref):
    xb = x_ref[...]
    g = lax.dot_general(xb, sg_ref[...], (((1,), (1,)), ((), ())),
                        preferred_element_type=_f32)
    u = lax.dot_general(xb, su_ref[...], (((1,), (1,)), ((), ())),
                        preferred_element_type=_f32)
    h = g * jax.nn.sigmoid(g) * u
    out_ref[...] = lax.dot_general(h, sd_ref[...], (((1,), (1,)), ((), ())),
                                   preferred_element_type=_f32)


def _sffn_call(x, s_gate, s_up, s_down):
    return pl.pallas_call(
        _sffn_body,
        grid=(T // _SBT,),
        in_specs=[
            pl.BlockSpec((_SBT, D), lambda t: (t, 0)),
            pl.BlockSpec((SDFF, D), lambda t: (0, 0)),
            pl.BlockSpec((SDFF, D), lambda t: (0, 0)),
            pl.BlockSpec((D, SDFF), lambda t: (0, 0)),
        ],
        out_specs=pl.BlockSpec((_SBT, D), lambda t: (t, 0)),
        out_shape=jax.ShapeDtypeStruct((T, D), _f32),
        compiler_params=pltpu.CompilerParams(
            dimension_semantics=("arbitrary",)),
    )(x, s_gate, s_up, s_down)


@functools.cache
def _build_gather():
    mesh = plsc.VectorSubcoreMesh(core_axis_name="c", subcore_axis_name="s",
                                  num_cores=2, num_subcores=16)

    @functools.partial(
        pl.kernel,
        out_type=jax.ShapeDtypeStruct((NPAIR, D), _f32),
        mesh=mesh,
        scratch_types=[
            pltpu.VMEM((2, 64), _i32),
            pltpu.VMEM((64, D), _f32),
            pltpu.SemaphoreType.DMA,
        ],
    )
    def gather(ys_hbm, pos_hbm, out_hbm, posv, rows, sem):
        wid = lax.axis_index("s") * 2 + lax.axis_index("c")
        pltpu.sync_copy(pos_hbm.at[wid], posv)
        for j in range(2):
            pltpu.async_copy(ys_hbm.at[posv.at[j]], rows, sem).wait()
            pltpu.sync_copy(rows, out_hbm.at[pl.ds(wid * 128 + j * 64, 64)])

    return gather


def _gather_call(ys, pos_arr):
    return _build_gather()(ys, pos_arr)


_CBT = 512


def _cmb_body(wts_ref, sh_ref, y1_ref, y2_ref, out_ref):
    w1 = wts_ref[:, 0:1]
    w2 = wts_ref[:, 1:2]
    out_ref[...] = sh_ref[...] + w1 * y1_ref[...] + w2 * y2_ref[...]


def _cmb_call(wts, sh, y1, y2):
    return pl.pallas_call(
        _cmb_body,
        grid=(T // _CBT,),
        in_specs=[
            pl.BlockSpec((_CBT, E), lambda t: (t, 0)),
            pl.BlockSpec((_CBT, D), lambda t: (t, 0)),
            pl.BlockSpec((_CBT, D), lambda t: (t, 0)),
            pl.BlockSpec((_CBT, D), lambda t: (t, 0)),
        ],
        out_specs=pl.BlockSpec((_CBT, D), lambda t: (t, 0)),
        out_shape=jax.ShapeDtypeStruct((T, D), _f32),
        compiler_params=pltpu.CompilerParams(
            dimension_semantics=("parallel",)),
    )(wts, sh, y1, y2)


def kernel(hidden_states, router_w, w_gate, w_up, w_down, s_gate, s_up,
           s_down):
    bsz, seq_len, h = hidden_states.shape
    x = hidden_states.reshape(T, D)

    logits = x @ router_w.T
    res, bexp = _plan_call(logits)
    w1 = res[:, 0]
    w2 = res[:, 1]
    pos1 = res[:, 2].astype(_i32)
    pos2 = res[:, 3].astype(_i32)
    be = bexp[0, :NB + 1].astype(_i32)

    tok = jnp.concatenate([jnp.arange(T, dtype=_i32),
                           jnp.arange(T, dtype=_i32)])
    pos = jnp.concatenate([pos1, pos2])
    xs = _dispatch_call(x, tok.reshape(NW, 2, 64), pos.reshape(NW, 2, 64))

    ys = _gffn_call(be, xs, w_gate, w_up, w_down)
    sh = _sffn_call(x, s_gate, s_up, s_down)

    yp = _gather_call(ys, pos.reshape(NW, 2, 64))
    y = _cmb_call(res, sh, yp[:T], yp[T:])
    return y.reshape(bsz, seq_len, h)


# speedup vs baseline: 1.2279x; 1.0152x over previous
import functools

import numpy as np

import jax
import jax.numpy as jnp
from jax import lax
from jax.experimental import pallas as pl
from jax.experimental.pallas import tpu as pltpu
from jax.experimental.pallas import tpu_sc as plsc

E = 8
TOPK = 2
D = 1024
DFF = 1024
SDFF = 2048
T = 2048
NPAIR = T * TOPK
BLK = 256
PMAX = NPAIR + E * BLK
NB = PMAX // BLK
NW = 32
BEXP_COLS = 64

_f32 = jnp.float32
_i32 = jnp.int32

_TOKTAB = np.tile(np.arange(T, dtype=np.int32), 2).reshape(NW, 2, 64)


def _plan_body(logits_ref, res_ref, pos_ref, bexp_ref):
    logits = logits_ref[...]
    m = jnp.max(logits, axis=1, keepdims=True)
    p = jnp.exp(logits - m)
    scores = p / jnp.sum(p, axis=1, keepdims=True)

    eidx = lax.broadcasted_iota(_i32, (T, E), 1).astype(_f32)
    m1 = jnp.max(scores, axis=1, keepdims=True)
    i1 = jnp.min(jnp.where(scores >= m1, eidx, float(E)), axis=1,
                 keepdims=True)
    c1 = (eidx == i1).astype(_f32)
    rest = jnp.where(c1 > 0, -jnp.inf, scores)
    m2 = jnp.max(rest, axis=1, keepdims=True)
    i2 = jnp.min(jnp.where(rest >= m2, eidx, float(E)), axis=1,
                 keepdims=True)
    c2 = (eidx == i2).astype(_f32)

    denom = m1 + m2 + 1e-20
    w1 = m1 / denom
    w2 = m2 / denom

    ones_t = jnp.ones((T, 1), _f32)
    ct = c1 + c2
    counts_col = lax.dot_general(ct, ones_t, (((0,), (0,)), ((), ())),
                                 precision=lax.Precision.HIGHEST)
    rup_col = jnp.ceil(counts_col / BLK) * BLK
    l8 = (lax.broadcasted_iota(_i32, (E, E), 1)
          < lax.broadcasted_iota(_i32, (E, E), 0)).astype(_f32)
    astart_col = lax.dot_general(l8, rup_col, (((1,), (0,)), ((), ())),
                                 precision=lax.Precision.HIGHEST)
    aend_col = astart_col + rup_col

    def rowb(v_col):
        return lax.dot_general(ones_t, v_col, (((1,), (1,)), ((), ())),
                               precision=lax.Precision.HIGHEST)

    ltri = (lax.broadcasted_iota(_i32, (T, T), 1)
            < lax.broadcasted_iota(_i32, (T, T), 0)).astype(_f32)
    cnt1 = lax.dot_general(ltri, c1, (((1,), (0,)), ((), ())))
    cnt2 = lax.dot_general(ltri, c2, (((1,), (0,)), ((), ())))
    c1tot_col = lax.dot_general(c1, ones_t, (((0,), (0,)), ((), ())),
                                precision=lax.Precision.HIGHEST)

    base = rowb(astart_col)
    pos1 = jnp.sum(c1 * (base + cnt1), axis=1, keepdims=True)
    pos2 = jnp.sum(c2 * (base + rowb(c1tot_col) + cnt2), axis=1,
                   keepdims=True)

    zero = jnp.zeros((T, E), _f32)
    res = (jnp.where(eidx == 0, w1, zero)
           + jnp.where(eidx == 1, w2, zero))
    res_ref[...] = res

    pos_ref[...] = jnp.concatenate(
        [jnp.reshape(pos1, (T // 128, 128)),
         jnp.reshape(pos2, (T // 128, 128))], axis=0).astype(_i32)

    ivals = lax.broadcasted_iota(_i32, (1, BEXP_COLS), 1).astype(_f32) * BLK
    ind = (ivals >= aend_col).astype(_f32)
    be_row = jnp.sum(ind, axis=0, keepdims=True)
    last_e = jnp.max(
        jnp.where(counts_col > 0,
                  lax.broadcasted_iota(_i32, (E, 1), 0).astype(_f32), 0.0))
    used = jnp.sum(rup_col) / BLK
    be_clamped = jnp.minimum(be_row, last_e)
    colid = lax.broadcasted_iota(_i32, (1, BEXP_COLS), 1).astype(_f32)
    bexp_ref[...] = jnp.where(colid == NB, used, be_clamped).astype(_i32)


def _plan_call(logits):
    return pl.pallas_call(
        _plan_body,
        out_shape=[
            jax.ShapeDtypeStruct((T, E), _f32),
            jax.ShapeDtypeStruct((NPAIR // 128, 128), _i32),
            jax.ShapeDtypeStruct((1, BEXP_COLS), _i32),
        ],
    )(logits)


@functools.cache
def _build_dispatch():
    mesh = plsc.VectorSubcoreMesh(core_axis_name="c", subcore_axis_name="s",
                                  num_cores=2, num_subcores=16)

    @functools.partial(
        pl.kernel,
        out_type=jax.ShapeDtypeStruct((PMAX, D), _f32),
        mesh=mesh,
        scratch_types=[
            pltpu.VMEM((2, 64), _i32),
            pltpu.VMEM((2, 64), _i32),
            pltpu.VMEM((64, D), _f32),
            pltpu.SemaphoreType.DMA,
            pltpu.SemaphoreType.DMA,
        ],
    )
    def dispatch(x_hbm, tok_hbm, pos_hbm, xs_hbm, tokv, posv, rows,
                 sem_g, sem_s):
        wid = lax.axis_index("s") * 2 + lax.axis_index("c")
        pltpu.sync_copy(tok_hbm.at[wid], tokv)
        pltpu.sync_copy(pos_hbm.at[wid], posv)
        for j in range(2):
            pltpu.async_copy(x_hbm.at[tokv.at[j]], rows, sem_g).wait()
            pltpu.async_copy(rows, xs_hbm.at[posv.at[j]], sem_s).wait()

    return dispatch


def _dispatch_call(x, tok_arr, pos_arr):
    return _build_dispatch()(x, tok_arr, pos_arr)


def _gffn_body(be_ref, xs_ref, wg_ref, wu_ref, wd_ref, out_ref):
    i = pl.program_id(0)

    @pl.when(i < be_ref[NB])
    def _():
        xb = xs_ref[...]
        g = lax.dot_general(xb, wg_ref[0], (((1,), (1,)), ((), ())),
                            preferred_element_type=_f32)
        u = lax.dot_general(xb, wu_ref[0], (((1,), (1,)), ((), ())),
                            preferred_element_type=_f32)
        h = g * jax.nn.sigmoid(g) * u
        out_ref[...] = lax.dot_general(h, wd_ref[0], (((1,), (1,)), ((), ())),
                                       preferred_element_type=_f32)


def _gffn_call(be, xs, w_gate, w_up, w_down):
    grid_spec = pltpu.PrefetchScalarGridSpec(
        num_scalar_prefetch=1,
        grid=(NB,),
        in_specs=[
            pl.BlockSpec((BLK, D), lambda i, be: (i, 0)),
            pl.BlockSpec((1, DFF, D), lambda i, be: (be[i], 0, 0)),
            pl.BlockSpec((1, DFF, D), lambda i, be: (be[i], 0, 0)),
            pl.BlockSpec((1, D, DFF), lambda i, be: (be[i], 0, 0)),
        ],
        out_specs=pl.BlockSpec((BLK, D), lambda i, be: (i, 0)),
    )
    return pl.pallas_call(
        _gffn_body,
        grid_spec=grid_spec,
        out_shape=jax.ShapeDtypeStruct((PMAX, D), _f32),
        compiler_params=pltpu.CompilerParams(
            dimension_semantics=("arbitrary",)),
    )(be, xs, w_gate, w_up, w_down)


_SBT = 512


def _sffn_body(x_ref, sg_ref, su_ref, sd_ref, out_ref):
    xb = x_ref[...]
    g = lax.dot_general(xb, sg_ref[...], (((1,), (1,)), ((), ())),
                        preferred_element_type=_f32)
    u = lax.dot_general(xb, su_ref[...], (((1,), (1,)), ((), ())),
                        preferred_element_type=_f32)
    h = g * jax.nn.sigmoid(g) * u
    out_ref[...] = lax.dot_general(h, sd_ref[...], (((1,), (1,)), ((), ())),
                                   preferred_element_type=_f32)


def _sffn_call(x, s_gate, s_up, s_down):
    return pl.pallas_call(
        _sffn_body,
        grid=(T // _SBT,),
        in_specs=[
            pl.BlockSpec((_SBT, D), lambda t: (t, 0)),
            pl.BlockSpec((SDFF, D), lambda t: (0, 0)),
            pl.BlockSpec((SDFF, D), lambda t: (0, 0)),
            pl.BlockSpec((D, SDFF), lambda t: (0, 0)),
        ],
        out_specs=pl.BlockSpec((_SBT, D), lambda t: (t, 0)),
        out_shape=jax.ShapeDtypeStruct((T, D), _f32),
        compiler_params=pltpu.CompilerParams(
            dimension_semantics=("arbitrary",)),
    )(x, s_gate, s_up, s_down)


@functools.cache
def _build_gather():
    mesh = plsc.VectorSubcoreMesh(core_axis_name="c", subcore_axis_name="s",
                                  num_cores=2, num_subcores=16)

    @functools.partial(
        pl.kernel,
        out_type=jax.ShapeDtypeStruct((NPAIR, D), _f32),
        mesh=mesh,
        scratch_types=[
            pltpu.VMEM((2, 64), _i32),
            pltpu.VMEM((64, D), _f32),
            pltpu.SemaphoreType.DMA,
        ],
    )
    def gather(ys_hbm, pos_hbm, out_hbm, posv, rows, sem):
        wid = lax.axis_index("s") * 2 + lax.axis_index("c")
        pltpu.sync_copy(pos_hbm.at[wid], posv)
        for j in range(2):
            pltpu.async_copy(ys_hbm.at[posv.at[j]], rows, sem).wait()
            pltpu.sync_copy(rows, out_hbm.at[pl.ds(wid * 128 + j * 64, 64)])

    return gather


def _gather_call(ys, pos_arr):
    return _build_gather()(ys, pos_arr)


_CBT = 512


def _cmb_body(wts_ref, sh_ref, y1_ref, y2_ref, out_ref):
    w1 = wts_ref[:, 0:1]
    w2 = wts_ref[:, 1:2]
    out_ref[...] = sh_ref[...] + w1 * y1_ref[...] + w2 * y2_ref[...]


def _cmb_call(wts, sh, y1, y2):
    return pl.pallas_call(
        _cmb_body,
        grid=(T // _CBT,),
        in_specs=[
            pl.BlockSpec((_CBT, E), lambda t: (t, 0)),
            pl.BlockSpec((_CBT, D), lambda t: (t, 0)),
            pl.BlockSpec((_CBT, D), lambda t: (t, 0)),
            pl.BlockSpec((_CBT, D), lambda t: (t, 0)),
        ],
        out_specs=pl.BlockSpec((_CBT, D), lambda t: (t, 0)),
        out_shape=jax.ShapeDtypeStruct((T, D), _f32),
        compiler_params=pltpu.CompilerParams(
            dimension_semantics=("parallel",)),
    )(wts, sh, y1, y2)


def kernel(hidden_states, router_w, w_gate, w_up, w_down, s_gate, s_up,
           s_down):
    bsz, seq_len, h = hidden_states.shape
    x = hidden_states.reshape(T, D)

    logits = x @ router_w.T
    res, posp, bexp = _plan_call(logits)
    be = bexp[0, :NB + 1]
    pos_arr = posp.reshape(NW, 2, 64)

    xs = _dispatch_call(x, jnp.asarray(_TOKTAB), pos_arr)
    sh = _sffn_call(x, s_gate, s_up, s_down)

    ys = _gffn_call(be, xs, w_gate, w_up, w_down)

    yp = _gather_call(ys, pos_arr)
    y = _cmb_call(res, sh, yp[:T], yp[T:])
    return y.reshape(bsz, seq_len, h)
